# Initial kernel scaffold; baseline (speedup 1.0000x reference)
import jax
import jax.numpy as jnp
from jax.experimental import pallas as pl


def kernel(x, edge_index, W, a):
    raise NotImplementedError("write your pallas kernel here")


# speedup vs baseline: 8.3100x; 8.3100x over previous
import jax
import jax.numpy as jnp
from jax import lax
from jax.experimental import pallas as pl
from jax.experimental.pallas import tpu as pltpu
from jax.experimental.pallas import tpu_sc as plsc

N = 10000
E = 320000
D = 128
DAUG = 144
NEG_SLOPE = 0.2

NC = 2
NS = 16
NW = NC * NS
EDGES_PER_W = E // NW
CHUNK = 80
NCHUNK = EDGES_PER_W // CHUNK
ROWS_PER_TILE = N // NS


def _prep_body(x_ref, w_ref, a_ref, haug_ref, s1_ref, s2_ref):
    h = jnp.dot(x_ref[...], w_ref[...], preferred_element_type=jnp.float32)
    haug_ref[:, :D] = h
    col = lax.broadcasted_iota(jnp.int32, (N, DAUG - D), 1)
    haug_ref[:, D:] = jnp.where(col == 0, 1.0, 0.0).astype(jnp.float32)
    s1_ref[...] = jnp.dot(h, a_ref[0, :D], preferred_element_type=jnp.float32)
    s2_ref[...] = jnp.dot(h, a_ref[0, D:], preferred_element_type=jnp.float32)


def _sc_body(haug, s1, s2, srcs, dsts, part,
             acc, src_v, dst_v, w_v, s1g, s2g, rows_v, sem_r, sem_1, sem_2):
    cid = lax.axis_index("c")
    sid = lax.axis_index("s")
    wid = cid * NS + sid

    zv = jnp.zeros((16,), jnp.float32)

    def zrow(r, carry):
        for j in range(DAUG // 16):
            rows_v[r, pl.ds(j * 16, 16)] = zv
        return carry

    lax.fori_loop(0, CHUNK, zrow, 0)
    base = sid * ROWS_PER_TILE
    for k in range(ROWS_PER_TILE // CHUNK):
        pltpu.sync_copy(rows_v, acc.at[pl.ds(base + k * CHUNK, CHUNK)])
    rem = ROWS_PER_TILE % CHUNK
    if rem:
        pltpu.sync_copy(rows_v.at[pl.ds(0, rem)],
                        acc.at[pl.ds(base + (ROWS_PER_TILE // CHUNK) * CHUNK, rem)])

    pltpu.sync_copy(srcs.at[wid], src_v)
    pltpu.sync_copy(dsts.at[wid], dst_v)

    plsc.subcore_barrier()

    def chunk_body(c, carry):
        cp_r = pltpu.async_copy(haug.at[dst_v.at[c]], rows_v, sem_r)
        cp_1 = pltpu.async_copy(s1.at[src_v.at[c]], s1g, sem_1)
        cp_2 = pltpu.async_copy(s2.at[dst_v.at[c]], s2g, sem_2)
        cp_1.wait()
        cp_2.wait()

        for i in range(CHUNK // 16):
            logit = s1g[pl.ds(i * 16, 16)] + s2g[pl.ds(i * 16, 16)]
            w = jnp.exp(jnp.where(logit > 0.0, -logit, (-NEG_SLOPE) * logit))
            w_v[pl.ds(i * 16, 16)] = w

        cp_r.wait()

        def scale(e, carry2):
            wv = plsc.load_gather(w_v, [jnp.broadcast_to(e, (16,)).astype(jnp.int32)])
            for j in range(DAUG // 16):
                rows_v[e, pl.ds(j * 16, 16)] = rows_v[e, pl.ds(j * 16, 16)] * wv
            return carry2

        lax.fori_loop(0, CHUNK, scale, 0)

        pltpu.sync_copy(rows_v, acc.at[src_v.at[c]], add=True)
        return carry

    lax.fori_loop(0, NCHUNK, chunk_body, 0)

    plsc.subcore_barrier()
    pltpu.sync_copy(acc.at[pl.ds(base, ROWS_PER_TILE)],
                    part.at[cid, pl.ds(base, ROWS_PER_TILE)])


_sc_call = pl.kernel(
    _sc_body,
    out_type=jax.ShapeDtypeStruct((NC, N, DAUG), jnp.float32),
    mesh=plsc.VectorSubcoreMesh(core_axis_name="c", subcore_axis_name="s",
                                num_cores=NC, num_subcores=NS),
    compiler_params=pltpu.CompilerParams(use_tc_tiling_on_sc=False,
                                         needs_layout_passes=False),
    scratch_types=[
        pltpu.VMEM_SHARED((N, DAUG), jnp.float32),
        pltpu.VMEM((NCHUNK, CHUNK), jnp.int32),
        pltpu.VMEM((NCHUNK, CHUNK), jnp.int32),
        pltpu.VMEM((CHUNK,), jnp.float32),
        pltpu.VMEM((CHUNK,), jnp.float32),
        pltpu.VMEM((CHUNK,), jnp.float32),
        pltpu.VMEM((CHUNK, DAUG), jnp.float32),
        pltpu.SemaphoreType.DMA,
        pltpu.SemaphoreType.DMA,
        pltpu.SemaphoreType.DMA,
    ],
)


def _fin_body(p_ref, o_ref):
    p = p_ref[0] + p_ref[1]
    hp = p[:, :D] / p[:, D:D + 1]
    o_ref[...] = jnp.where(hp > 0.0, hp, jnp.exp(hp) - 1.0)


def kernel(x, edge_index, W, a):
    src = edge_index[0].astype(jnp.int32).reshape(NW, NCHUNK, CHUNK)
    dst = edge_index[1].astype(jnp.int32).reshape(NW, NCHUNK, CHUNK)
    haug, s1, s2 = pl.pallas_call(
        _prep_body,
        out_shape=(jax.ShapeDtypeStruct((N, DAUG), jnp.float32),
                   jax.ShapeDtypeStruct((N,), jnp.float32),
                   jax.ShapeDtypeStruct((N,), jnp.float32)),
    )(x, W, a)
    part = _sc_call(haug, s1, s2, src, dst)
    return pl.pallas_call(
        _fin_body,
        out_shape=jax.ShapeDtypeStruct((N, D), jnp.float32),
    )(part)


# speedup vs baseline: 12.1996x; 1.4681x over previous
import jax
import jax.numpy as jnp
from jax import lax
from jax.experimental import pallas as pl
from jax.experimental.pallas import tpu as pltpu
from jax.experimental.pallas import tpu_sc as plsc

N = 10000
E = 320000
D = 128
DAUG = 144
NEG_SLOPE = 0.2

NC = 2
NS = 16
NW = NC * NS
EDGES_PER_W = E // NW
CHUNK = 80
NCHUNK = EDGES_PER_W // CHUNK
NPAIR = NCHUNK // 2
ROWS_PER_TILE = N // NS


def _prep_body(x_ref, w_ref, a_ref, haug_ref, s1_ref, s2_ref):
    h = jnp.dot(x_ref[...], w_ref[...], preferred_element_type=jnp.float32)
    haug_ref[:, :D] = h
    col = lax.broadcasted_iota(jnp.int32, (N, DAUG - D), 1)
    haug_ref[:, D:] = jnp.where(col == 0, 1.0, 0.0).astype(jnp.float32)
    s1_ref[...] = jnp.dot(h, a_ref[0, :D], preferred_element_type=jnp.float32)
    s2_ref[...] = jnp.dot(h, a_ref[0, D:], preferred_element_type=jnp.float32)


def _sc_body(haug, s1, s2, srcs, dsts, part,
             acc, srcc0, srcc1, dstc0, dstc1, w_v, s1g0, s1g1, s2g0, s2g1,
             rows0, rows1, sem_i0, sem_i1, sem_r0, sem_r1,
             sem_10, sem_11, sem_20, sem_21):
    cid = lax.axis_index("c")
    sid = lax.axis_index("s")
    wid = cid * NS + sid

    srcc = (srcc0, srcc1)
    dstc = (dstc0, dstc1)
    s1g = (s1g0, s1g1)
    s2g = (s2g0, s2g1)
    rows = (rows0, rows1)
    sem_i = (sem_i0, sem_i1)
    sem_r = (sem_r0, sem_r1)
    sem_1 = (sem_10, sem_11)
    sem_2 = (sem_20, sem_21)

    zv = jnp.zeros((16,), jnp.float32)

    def zrow(r, carry):
        for j in range(DAUG // 16):
            rows0[r, pl.ds(j * 16, 16)] = zv
        return carry

    lax.fori_loop(0, CHUNK, zrow, 0)
    base = sid * ROWS_PER_TILE
    for k in range(ROWS_PER_TILE // CHUNK):
        pltpu.sync_copy(rows0, acc.at[pl.ds(base + k * CHUNK, CHUNK)])
    rem = ROWS_PER_TILE % CHUNK
    if rem:
        pltpu.sync_copy(rows0.at[pl.ds(0, rem)],
                        acc.at[pl.ds(base + (ROWS_PER_TILE // CHUNK) * CHUNK, rem)])

    plsc.subcore_barrier()

    def start_idx(c, b):
        pltpu.async_copy(srcs.at[wid, c], srcc[b], sem_i[b])
        pltpu.async_copy(dsts.at[wid, c], dstc[b], sem_i[b])

    def wait_idx(b):
        pltpu.make_async_copy(srcs.at[wid, 0], srcc[b], sem_i[b]).wait()
        pltpu.make_async_copy(dsts.at[wid, 0], dstc[b], sem_i[b]).wait()

    def start_gathers(b):
        pltpu.async_copy(haug.at[dstc[b]], rows[b], sem_r[b])
        pltpu.async_copy(s1.at[srcc[b]], s1g[b], sem_1[b])
        pltpu.async_copy(s2.at[dstc[b]], s2g[b], sem_2[b])

    def compute_chunk(b):
        pltpu.make_async_copy(s1.at[srcc[b]], s1g[b], sem_1[b]).wait()
        pltpu.make_async_copy(s2.at[dstc[b]], s2g[b], sem_2[b]).wait()
        for i in range(CHUNK // 16):
            logit = s1g[b][pl.ds(i * 16, 16)] + s2g[b][pl.ds(i * 16, 16)]
            w = jnp.exp(jnp.where(logit > 0.0, -logit, (-NEG_SLOPE) * logit))
            w_v[pl.ds(i * 16, 16)] = w

        pltpu.make_async_copy(haug.at[dstc[b]], rows[b], sem_r[b]).wait()

        @plsc.parallel_loop(0, CHUNK, unroll=4)
        def scale(e):
            wv = plsc.load_gather(w_v, [jnp.broadcast_to(e, (16,)).astype(jnp.int32)])
            for j in range(DAUG // 16):
                rows[b][e, pl.ds(j * 16, 16)] = rows[b][e, pl.ds(j * 16, 16)] * wv

        pltpu.sync_copy(rows[b], acc.at[srcc[b]], add=True)

    start_idx(0, 0)
    start_idx(1, 1)
    wait_idx(0)
    start_gathers(0)

    def pair_body(c0, carry):
        for b in range(2):
            c = 2 * c0 + b
            nb = 1 - b
            wait_idx(nb)
            start_gathers(nb)
            compute_chunk(b)
            start_idx(jnp.minimum(c + 2, NCHUNK - 1), b)
        return carry

    lax.fori_loop(0, NPAIR, pair_body, 0)

    compute_chunk(0)
    wait_idx(1)

    plsc.subcore_barrier()
    pltpu.sync_copy(acc.at[pl.ds(base, ROWS_PER_TILE)],
                    part.at[cid, pl.ds(base, ROWS_PER_TILE)])


_sc_call = pl.kernel(
    _sc_body,
    out_type=jax.ShapeDtypeStruct((NC, N, DAUG), jnp.float32),
    mesh=plsc.VectorSubcoreMesh(core_axis_name="c", subcore_axis_name="s",
                                num_cores=NC, num_subcores=NS),
    compiler_params=pltpu.CompilerParams(use_tc_tiling_on_sc=False,
                                         needs_layout_passes=False),
    scratch_types=[
        pltpu.VMEM_SHARED((N, DAUG), jnp.float32),
        pltpu.VMEM((CHUNK,), jnp.int32),
        pltpu.VMEM((CHUNK,), jnp.int32),
        pltpu.VMEM((CHUNK,), jnp.int32),
        pltpu.VMEM((CHUNK,), jnp.int32),
        pltpu.VMEM((CHUNK,), jnp.float32),
        pltpu.VMEM((CHUNK,), jnp.float32),
        pltpu.VMEM((CHUNK,), jnp.float32),
        pltpu.VMEM((CHUNK,), jnp.float32),
        pltpu.VMEM((CHUNK,), jnp.float32),
        pltpu.VMEM((CHUNK, DAUG), jnp.float32),
        pltpu.VMEM((CHUNK, DAUG), jnp.float32),
        pltpu.SemaphoreType.DMA,
        pltpu.SemaphoreType.DMA,
        pltpu.SemaphoreType.DMA,
        pltpu.SemaphoreType.DMA,
        pltpu.SemaphoreType.DMA,
        pltpu.SemaphoreType.DMA,
        pltpu.SemaphoreType.DMA,
        pltpu.SemaphoreType.DMA,
    ],
)


def _fin_body(p_ref, o_ref):
    p = p_ref[0] + p_ref[1]
    hp = p[:, :D] / p[:, D:D + 1]
    o_ref[...] = jnp.where(hp > 0.0, hp, jnp.exp(hp) - 1.0)


def kernel(x, edge_index, W, a):
    src = edge_index[0].astype(jnp.int32).reshape(NW, NCHUNK, CHUNK)
    dst = edge_index[1].astype(jnp.int32).reshape(NW, NCHUNK, CHUNK)
    haug, s1, s2 = pl.pallas_call(
        _prep_body,
        out_shape=(jax.ShapeDtypeStruct((N, DAUG), jnp.float32),
                   jax.ShapeDtypeStruct((N,), jnp.float32),
                   jax.ShapeDtypeStruct((N,), jnp.float32)),
    )(x, W, a)
    part = _sc_call(haug, s1, s2, src, dst)
    return pl.pallas_call(
        _fin_body,
        out_shape=jax.ShapeDtypeStruct((N, D), jnp.float32),
    )(part)


# speedup vs baseline: 17.2950x; 1.4177x over previous
import jax
import jax.numpy as jnp
from jax import lax
from jax.experimental import pallas as pl
from jax.experimental.pallas import tpu as pltpu
from jax.experimental.pallas import tpu_sc as plsc

N = 10000
E = 320000
D = 128
NEG_SLOPE = 0.2

NC = 2
NS = 16
NW = NC * NS
EDGES_PER_W = E // NW
CHUNK = 100
CHUNK_PAD = 112
NCHUNK = EDGES_PER_W // CHUNK
NPAIR = NCHUNK // 2 - 1
ROWS_PER_TILE = N // NS
NSUM = 10240


def _prep_body(x_ref, w_ref, a_ref, h_ref, s1_ref, s2_ref):
    h = jnp.dot(x_ref[...], w_ref[...], preferred_element_type=jnp.float32)
    h_ref[...] = h
    s1_ref[...] = jnp.dot(h, a_ref[0, :D], preferred_element_type=jnp.float32)
    s2_ref[...] = jnp.dot(h, a_ref[0, D:], preferred_element_type=jnp.float32)


def _sc_body(h, s1, s2, srcs, dsts, part, psum,
             acc, acc1, src_v, dst_v, w_v, s1g0, s1g1, s2g0, s2g1,
             rows0, rows1, zb1, sem_r0, sem_r1, sem_10, sem_11, sem_20, sem_21):
    cid = lax.axis_index("c")
    sid = lax.axis_index("s")
    wid = cid * NS + sid

    s1g = (s1g0, s1g1)
    s2g = (s2g0, s2g1)
    rows = (rows0, rows1)
    sem_r = (sem_r0, sem_r1)
    sem_1 = (sem_10, sem_11)
    sem_2 = (sem_20, sem_21)

    zv = jnp.zeros((16,), jnp.float32)

    def zrow(r, carry):
        for j in range(D // 16):
            rows0[r, pl.ds(j * 16, 16)] = zv
        return carry

    lax.fori_loop(0, CHUNK, zrow, 0)
    for i in range(NSUM // NS // 16):
        zb1[pl.ds(i * 16, 16)] = zv
    base = sid * ROWS_PER_TILE
    for k in range(ROWS_PER_TILE // CHUNK):
        pltpu.sync_copy(rows0, acc.at[pl.ds(base + k * CHUNK, CHUNK)])
    rem = ROWS_PER_TILE % CHUNK
    if rem:
        pltpu.sync_copy(rows0.at[pl.ds(0, rem)],
                        acc.at[pl.ds(base + (ROWS_PER_TILE // CHUNK) * CHUNK, rem)])
    pltpu.sync_copy(zb1, acc1.at[pl.ds(sid * (NSUM // NS), NSUM // NS)])

    pltpu.sync_copy(srcs.at[wid], src_v)
    pltpu.sync_copy(dsts.at[wid], dst_v)

    plsc.subcore_barrier()

    def start_gathers(c, b):
        pltpu.async_copy(h.at[dst_v.at[c]], rows[b], sem_r[b])
        pltpu.async_copy(s1.at[src_v.at[c]], s1g[b].at[pl.ds(0, CHUNK)], sem_1[b])
        pltpu.async_copy(s2.at[dst_v.at[c]], s2g[b].at[pl.ds(0, CHUNK)], sem_2[b])

    def compute_chunk(c, b):
        pltpu.make_async_copy(s1.at[src_v.at[c]], s1g[b].at[pl.ds(0, CHUNK)],
                              sem_1[b]).wait()
        pltpu.make_async_copy(s2.at[dst_v.at[c]], s2g[b].at[pl.ds(0, CHUNK)],
                              sem_2[b]).wait()
        for i in range(CHUNK_PAD // 16):
            logit = s1g[b][pl.ds(i * 16, 16)] + s2g[b][pl.ds(i * 16, 16)]
            w = jnp.exp(jnp.where(logit > 0.0, -logit, (-NEG_SLOPE) * logit))
            w_v[pl.ds(i * 16, 16)] = w

        pltpu.make_async_copy(h.at[dst_v.at[c]], rows[b], sem_r[b]).wait()

        @plsc.parallel_loop(0, CHUNK, unroll=4)
        def scale(e):
            wv = plsc.load_gather(w_v, [jnp.broadcast_to(e, (16,)).astype(jnp.int32)])
            for j in range(D // 16):
                rows[b][e, pl.ds(j * 16, 16)] = rows[b][e, pl.ds(j * 16, 16)] * wv

        pltpu.sync_copy(w_v.at[pl.ds(0, CHUNK)], acc1.at[src_v.at[c]], add=True)
        pltpu.sync_copy(rows[b], acc.at[src_v.at[c]], add=True)

    start_gathers(0, 0)

    def pair_body(c0, carry):
        c = 2 * c0
        start_gathers(c + 1, 1)
        compute_chunk(c, 0)
        start_gathers(c + 2, 0)
        compute_chunk(c + 1, 1)
        return carry

    lax.fori_loop(0, NPAIR, pair_body, 0)

    start_gathers(NCHUNK - 1, 1)
    compute_chunk(NCHUNK - 2, 0)
    compute_chunk(NCHUNK - 1, 1)

    plsc.subcore_barrier()
    pltpu.sync_copy(acc.at[pl.ds(base, ROWS_PER_TILE)],
                    part.at[cid, pl.ds(base, ROWS_PER_TILE)])
    pltpu.sync_copy(acc1.at[pl.ds(sid * (NSUM // NS), NSUM // NS)],
                    psum.at[cid, pl.ds(sid * (NSUM // NS), NSUM // NS)])


_sc_call = pl.kernel(
    _sc_body,
    out_type=(jax.ShapeDtypeStruct((NC, N, D), jnp.float32),
              jax.ShapeDtypeStruct((NC, NSUM), jnp.float32)),
    mesh=plsc.VectorSubcoreMesh(core_axis_name="c", subcore_axis_name="s",
                                num_cores=NC, num_subcores=NS),
    compiler_params=pltpu.CompilerParams(use_tc_tiling_on_sc=False,
                                         needs_layout_passes=False),
    scratch_types=[
        pltpu.VMEM_SHARED((N, D), jnp.float32),
        pltpu.VMEM_SHARED((NSUM,), jnp.float32),
        pltpu.VMEM((NCHUNK, CHUNK), jnp.int32),
        pltpu.VMEM((NCHUNK, CHUNK), jnp.int32),
        pltpu.VMEM((CHUNK_PAD,), jnp.float32),
        pltpu.VMEM((CHUNK_PAD,), jnp.float32),
        pltpu.VMEM((CHUNK_PAD,), jnp.float32),
        pltpu.VMEM((CHUNK_PAD,), jnp.float32),
        pltpu.VMEM((CHUNK_PAD,), jnp.float32),
        pltpu.VMEM((CHUNK, D), jnp.float32),
        pltpu.VMEM((CHUNK, D), jnp.float32),
        pltpu.VMEM((NSUM // NS,), jnp.float32),
        pltpu.SemaphoreType.DMA,
        pltpu.SemaphoreType.DMA,
        pltpu.SemaphoreType.DMA,
        pltpu.SemaphoreType.DMA,
        pltpu.SemaphoreType.DMA,
        pltpu.SemaphoreType.DMA,
    ],
)


def _fin_body(p_ref, r_ref, o_ref):
    p = p_ref[0] + p_ref[1]
    r = r_ref[0, :N] + r_ref[1, :N]
    hp = p / jnp.reshape(r, (N, 1))
    o_ref[...] = jnp.where(hp > 0.0, hp, jnp.exp(hp) - 1.0)


def kernel(x, edge_index, W, a):
    src = edge_index[0].astype(jnp.int32).reshape(NW, NCHUNK, CHUNK)
    dst = edge_index[1].astype(jnp.int32).reshape(NW, NCHUNK, CHUNK)
    h, s1, s2 = pl.pallas_call(
        _prep_body,
        out_shape=(jax.ShapeDtypeStruct((N, D), jnp.float32),
                   jax.ShapeDtypeStruct((N,), jnp.float32),
                   jax.ShapeDtypeStruct((N,), jnp.float32)),
    )(x, W, a)
    part, psum = _sc_call(h, s1, s2, src, dst)
    return pl.pallas_call(
        _fin_body,
        out_shape=jax.ShapeDtypeStruct((N, D), jnp.float32),
    )(part, psum)
